# baseline (speedup 1.0000x reference)
import functools

import jax
import jax.numpy as jnp
from jax import lax
from jax.experimental import pallas as pl
from jax.experimental.pallas import tpu as pltpu
from jax.experimental.pallas import tpu_sc as plsc

V = 1000000
D = 64
BX = 16384
SQ = 50
NC = 2
NS = 16
NW = NC * NS
BW = BX // NW
PV = V // 2
NTB = 7812
TPW = NTB // NW
CH = 128
NCH = SQ * (BW // CH)

_mesh = plsc.VectorSubcoreMesh(core_axis_name="c", subcore_axis_name="s")


def _iota16():
    return lax.iota(jnp.int32, 16)


def _transpose_block(src, dst, q0, nq):
    def tq(q, carry):
        for j in range(8):
            row = (j % 4) * 16 + _iota16()
            col = jnp.full((16,), 0, jnp.int32) + (2 * q + j // 4)
            dst[q, pl.ds(j * 16, 16)] = plsc.load_gather(src, [row, col])
        return carry

    lax.fori_loop(q0, q0 + nq, tq, 0)


def _trans_body(wt_hbm, tail_hbm, pk_hbm, ib0, ib1, tb0, tb1,
                isem0, isem1, osem0, osem1):
    wid = lax.axis_index("s") * NC + lax.axis_index("c")
    ib = (ib0, ib1)
    tb = (tb0, tb1)
    isem = (isem0, isem1)
    osem = (osem0, osem1)

    def tcol(t_local):
        return (wid + NW * t_local) * 128

    def start_in(t_local, b):
        pltpu.async_copy(wt_hbm.at[:, pl.ds(tcol(t_local), 128)], ib[b], isem[b])

    def wait_in(b):
        pltpu.make_async_copy(wt_hbm.at[:, pl.ds(0, 128)], ib[b], isem[b]).wait()

    def start_out(t_local, b):
        pltpu.async_copy(tb[b], pk_hbm.at[pl.ds((wid + NW * t_local) * 64, 64)], osem[b])

    def wait_out(b):
        pltpu.make_async_copy(tb[b], pk_hbm.at[pl.ds(0, 64)], osem[b]).wait()

    start_in(0, 0)
    start_in(1, 1)

    def body(i, carry):
        for b in range(2):
            t_local = i * 2 + b
            wait_in(b)

            @pl.when(t_local >= 2)
            def _():
                wait_out(b)

            _transpose_block(ib[b], tb[b], 0, 64)
            start_out(t_local, b)

            @pl.when(t_local < TPW - 2)
            def _():
                start_in(t_local + 2, b)

        return carry

    lax.fori_loop(0, TPW // 2, body, 0)
    wait_out(0)
    wait_out(1)

    @pl.when(wid < NTB - NW * TPW)
    def _():
        pltpu.sync_copy(wt_hbm.at[:, pl.ds((NW * TPW + wid) * 128, 128)], ib0)
        _transpose_block(ib0, tb0, 0, 64)
        pltpu.sync_copy(tb0, pk_hbm.at[pl.ds((NW * TPW + wid) * 64, 64)])

    @pl.when(wid == NW - 1)
    def _():
        pltpu.sync_copy(tail_hbm, ib1)
        _transpose_block(ib1, tb1, 0, (V - NTB * 128) // 2)
        pltpu.sync_copy(
            tb1.at[pl.ds(0, (V - NTB * 128) // 2)],
            pk_hbm.at[pl.ds(NTB * 64, (V - NTB * 128) // 2)])

    return


def _gather_body(xt_hbm, pk_hbm, out_hbm,
                 idx_v, hidx0, hidx1, gb0, gb1, ob0, ob1,
                 gsem0, gsem1, osem0, osem1):
    wid = lax.axis_index("s") * NC + lax.axis_index("c")
    cb = wid * BW
    hidx = (hidx0, hidx1)
    gb = (gb0, gb1)
    ob = (ob0, ob1)
    gsem = (gsem0, gsem1)
    osem = (osem0, osem1)

    pltpu.sync_copy(xt_hbm.at[:, pl.ds(cb, BW)], idx_v)

    def prep(j, b):
        s = lax.shift_right_logical(j, 2)
        c4 = (j & 3) * 128
        for g in range(CH // 16):
            v = idx_v[s, pl.ds(c4 + g * 16, 16)]
            hidx[b][pl.ds(g * 16, 16)] = lax.shift_right_logical(v, 1)

    def start_gather(b):
        pltpu.async_copy(pk_hbm.at[hidx[b]], gb[b], gsem[b])

    def wait_gather(b):
        pltpu.make_async_copy(pk_hbm.at[hidx[b]], gb[b], gsem[b]).wait()

    def start_out(s, os):
        pltpu.async_copy(ob[os], out_hbm.at[s, :, pl.ds(cb, BW)], osem[os])

    def wait_out(os):
        pltpu.make_async_copy(ob[os], out_hbm.at[0, :, pl.ds(cb, BW)], osem[os]).wait()

    def transpose_chunk(s, c4, b, os):
        for g in range(CH // 16):
            v = idx_v[s, pl.ds(c4 * 128 + g * 16, 16)]
            colv = lax.shift_left(v & 1, 6)
            row = g * 16 + _iota16()

            def cblk(ci, carry):
                c0 = ci * 8
                for cc in range(8):
                    ob[os][c0 + cc, pl.ds(c4 * 128 + g * 16, 16)] = (
                        plsc.load_gather(gb[b], [row, colv + (c0 + cc)]))
                return carry

            lax.fori_loop(0, D // 8, cblk, 0)

    prep(0, 0)
    start_gather(0)
    prep(1, 1)
    start_gather(1)

    def body(sp, carry):
        for ph in range(2):
            s = sp * 2 + ph

            @pl.when(sp >= 1)
            def _():
                wait_out(ph)

            for c4 in range(4):
                j = s * 4 + c4
                b = c4 & 1
                wait_gather(b)
                transpose_chunk(s, c4, b, ph)
                nj = j + 2

                @pl.when(nj < NCH)
                def _():
                    prep(nj, b)
                    start_gather(b)

            start_out(s, ph)

        return carry

    lax.fori_loop(0, SQ // 2, body, 0)
    wait_out(0)
    wait_out(1)
    return


_trans_kernel = pl.kernel(
    _trans_body,
    mesh=_mesh,
    compiler_params=pltpu.CompilerParams(needs_layout_passes=False),
    out_type=jax.ShapeDtypeStruct((PV, 2 * D), jnp.float32),
    scratch_types=[
        pltpu.VMEM((D, 128), jnp.float32),
        pltpu.VMEM((D, 128), jnp.float32),
        pltpu.VMEM((D, 128), jnp.float32),
        pltpu.VMEM((D, 128), jnp.float32),
        pltpu.SemaphoreType.DMA,
        pltpu.SemaphoreType.DMA,
        pltpu.SemaphoreType.DMA,
        pltpu.SemaphoreType.DMA,
    ],
)

_gather_kernel = pl.kernel(
    _gather_body,
    mesh=_mesh,
    compiler_params=pltpu.CompilerParams(needs_layout_passes=False),
    out_type=jax.ShapeDtypeStruct((SQ, D, BX), jnp.float32),
    scratch_types=[
        pltpu.VMEM((SQ, BW), jnp.int32),
        pltpu.VMEM((CH,), jnp.int32),
        pltpu.VMEM((CH,), jnp.int32),
        pltpu.VMEM((CH, 2 * D), jnp.float32),
        pltpu.VMEM((CH, 2 * D), jnp.float32),
        pltpu.VMEM((D, BW), jnp.float32),
        pltpu.VMEM((D, BW), jnp.float32),
        pltpu.SemaphoreType.DMA,
        pltpu.SemaphoreType.DMA,
        pltpu.SemaphoreType.DMA,
        pltpu.SemaphoreType.DMA,
    ],
)


def kernel(x, weight):
    tail = jnp.pad(weight[NTB * 128:].T, ((0, 0), (0, 2 * D - (V - NTB * 128))))
    packed = _trans_kernel(weight.T, tail)
    yt = _gather_kernel(x.T, packed)
    return jnp.transpose(yt, (2, 0, 1))


# speedup vs baseline: 2.2049x; 2.2049x over previous
import functools

import jax
import jax.numpy as jnp
from jax import lax
from jax.experimental import pallas as pl
from jax.experimental.pallas import tpu as pltpu
from jax.experimental.pallas import tpu_sc as plsc

V = 1000000
D = 64
BX = 16384
SQ = 50
NC = 2
NS = 16
NW = NC * NS
BW = BX // NW
CH = 128
NG = BW // CH

_mesh = plsc.VectorSubcoreMesh(core_axis_name="c", subcore_axis_name="s")


def _gather_body(xt_hbm, table_hbm, out_hbm,
                 idx_v, ob0, ob1, gsem0, gsem1, osem0, osem1):
    wid = lax.axis_index("s") * NC + lax.axis_index("c")
    b0 = wid * BW
    ob = (ob0, ob1)
    gsem = (gsem0, gsem1)
    osem = (osem0, osem1)

    pltpu.sync_copy(xt_hbm.at[:, pl.ds(b0, BW)], idx_v)

    def fire_gathers(s, slot):
        for k in range(NG):
            pltpu.async_copy(
                table_hbm.at[idx_v.at[s, pl.ds(k * CH, CH)]],
                ob[slot].at[pl.ds(k * CH, CH)], gsem[slot])

    def wait_gathers(slot):
        for k in range(NG):
            pltpu.make_async_copy(
                table_hbm.at[idx_v.at[0, pl.ds(0, CH)]],
                ob[slot].at[pl.ds(k * CH, CH)], gsem[slot]).wait()

    def start_out(s, slot):
        pltpu.async_copy(ob[slot], out_hbm.at[pl.ds(b0, BW), s], osem[slot])

    def wait_out(slot):
        pltpu.make_async_copy(ob[slot], out_hbm.at[pl.ds(b0, BW), 0],
                              osem[slot]).wait()

    fire_gathers(0, 0)

    def body(sp, carry):
        for ph in range(2):
            s = sp * 2 + ph

            @pl.when(s >= 1)
            def _():
                wait_out(1 - ph)

            @pl.when(s + 1 < SQ)
            def _():
                fire_gathers(s + 1, 1 - ph)

            wait_gathers(ph)
            start_out(s, ph)

        return carry

    lax.fori_loop(0, SQ // 2, body, 0)
    wait_out(1)
    return


_gather_kernel = pl.kernel(
    _gather_body,
    mesh=_mesh,
    compiler_params=pltpu.CompilerParams(use_tc_tiling_on_sc=False),
    out_type=jax.ShapeDtypeStruct((BX, SQ, D), jnp.float32),
    scratch_types=[
        pltpu.VMEM((SQ, BW), jnp.int32),
        pltpu.VMEM((BW, D), jnp.float32),
        pltpu.VMEM((BW, D), jnp.float32),
        pltpu.SemaphoreType.DMA,
        pltpu.SemaphoreType.DMA,
        pltpu.SemaphoreType.DMA,
        pltpu.SemaphoreType.DMA,
    ],
)


def kernel(x, weight):
    return _gather_kernel(x.T, weight)
